# Initial kernel scaffold; baseline (speedup 1.0000x reference)
import jax
import jax.numpy as jnp
from jax.experimental import pallas as pl


def kernel(h, e, batch_llms, edge_index, W_h, b_h, pe_emb, W_pe, b_pe, Wl, bl, gamma, beta, W1, b1, Wg, bg, Wr0, br0, Wr1, br1, Wr2, br2):
    raise NotImplementedError("write your pallas kernel here")


# speedup vs baseline: 8.9960x; 8.9960x over previous
import functools

import jax
import jax.numpy as jnp
from jax import lax
from jax.experimental import pallas as pl
from jax.experimental.pallas import tpu as pltpu
from jax.experimental.pallas import tpu_sc as plsc

_N = 10000
_E = 320000
_H = 128
_B = 100
_NN = 100
_L = 4
_NC = 2
_NS = 16
_NW = _NC * _NS
_EPT = _E // _NS
_EPW = _E // _NW
_K = 80
_NCH = _EPW // _K
_RPT = _N // _NS
_NPAD = 10240
_HR = _NPAD // 16
_BLK = 2000
_G = _N // _BLK

_mesh = plsc.VectorSubcoreMesh(core_axis_name="c", subcore_axis_name="s")


_KD = 80
_NCHD = _EPT // _KD
_NPT = _NPAD // _NS


@functools.partial(
    pl.kernel,
    out_type=jax.ShapeDtypeStruct((_NC * _NPAD,), jnp.float32),
    mesh=_mesh,
    scratch_types=[
        pltpu.VMEM((_NCHD, _KD), jnp.int32),
        pltpu.VMEM((_KD,), jnp.float32),
        pltpu.VMEM((_NPT,), jnp.float32),
        pltpu.VMEM_SHARED((_NPAD,), jnp.float32),
    ],
)
def _deg_call(dst_src_hbm, deg_hbm, idx_v, ones_v, zb_v, acc_sh):
    c = lax.axis_index("c")
    s = lax.axis_index("s")
    zeros16 = jnp.zeros((16,), jnp.float32)
    ones16 = jnp.ones((16,), jnp.float32)

    def zb_body(i, _):
        zb_v[pl.ds(i * 16, 16)] = zeros16
        return 0

    lax.fori_loop(0, _NPT // 16, zb_body, 0)

    def ob_body(i, _):
        ones_v[pl.ds(i * 16, 16)] = ones16
        return 0

    lax.fori_loop(0, _KD // 16, ob_body, 0)

    pltpu.sync_copy(zb_v, acc_sh.at[pl.ds(s * _NPT, _NPT)])
    pltpu.sync_copy(dst_src_hbm.at[c, s], idx_v)
    plsc.subcore_barrier()

    def body(j, _):
        pltpu.sync_copy(ones_v, acc_sh.at[idx_v.at[j]], add=True)
        return 0

    lax.fori_loop(0, _NCHD, body, 0)
    plsc.subcore_barrier()
    base = pl.multiple_of(c * _NPAD + s * _NPT, 128)
    pltpu.sync_copy(
        acc_sh.at[pl.ds(s * _NPT, _NPT)],
        deg_hbm.at[pl.ds(base, _NPT)],
    )


@functools.partial(
    pl.kernel,
    out_type=jax.ShapeDtypeStruct((_NC * _NPAD, _H), jnp.float32),
    mesh=_mesh,
    scratch_types=[
        pltpu.VMEM((_NCH, _K), jnp.int32),
        pltpu.VMEM((_NCH, _K), jnp.int32),
        pltpu.VMEM((_K, _H), jnp.float32),
        pltpu.VMEM_SHARED((_NPAD, _H), jnp.float32),
    ],
)
def _spmm_call(hs_hbm, src_hbm, dst_hbm, zero_hbm, out_hbm, srcv, dstv, rows_v, agg_sh):
    c = lax.axis_index("c")
    s = lax.axis_index("s")
    wid = c * _NS + s
    pltpu.sync_copy(
        zero_hbm.at[pl.ds(s * _NPT, _NPT)], agg_sh.at[pl.ds(s * _NPT, _NPT)]
    )
    pltpu.sync_copy(src_hbm.at[wid], srcv)
    pltpu.sync_copy(dst_hbm.at[wid], dstv)
    plsc.subcore_barrier()

    def body(j, _):
        pltpu.sync_copy(hs_hbm.at[srcv.at[j]], rows_v)
        pltpu.sync_copy(rows_v, agg_sh.at[dstv.at[j]], add=True)
        return 0

    lax.fori_loop(0, _NCH, body, 0)
    plsc.subcore_barrier()
    base = pl.multiple_of(c * _NPAD + s * _NPT, 8)
    pltpu.sync_copy(
        agg_sh.at[pl.ds(s * _NPT, _NPT)],
        out_hbm.at[pl.ds(base, _NPT)],
    )


def _pe_body(pe_emb_ref, wpe_ref, bpe_ref, out_ref):
    out_ref[...] = (
        jnp.dot(pe_emb_ref[...], wpe_ref[...], preferred_element_type=jnp.float32)
        + bpe_ref[...]
    )


_pe_call = pl.pallas_call(
    _pe_body,
    out_shape=jax.ShapeDtypeStruct((_NN, _H), jnp.float32),
)


def _prep_body(h_ref, pef_ref, wh_ref, bh_ref, od_ref, h0_ref, hs_ref):
    h0 = (
        jnp.dot(h_ref[...], wh_ref[...], preferred_element_type=jnp.float32)
        + bh_ref[...]
        + pef_ref[...]
    )
    inv_out = lax.rsqrt(jnp.maximum(od_ref[...], 1.0))
    h0_ref[...] = h0
    hs_ref[...] = h0 * inv_out


_prep_call = pl.pallas_call(
    _prep_body,
    grid=(_G,),
    in_specs=[
        pl.BlockSpec((_BLK, _H), lambda g: (g, 0)),
        pl.BlockSpec((_BLK, _H), lambda g: (g, 0)),
        pl.BlockSpec((_H, _H), lambda g: (0, 0)),
        pl.BlockSpec((1, _H), lambda g: (0, 0)),
        pl.BlockSpec((_BLK, 1), lambda g: (g, 0)),
    ],
    out_specs=[
        pl.BlockSpec((_BLK, _H), lambda g: (g, 0)),
        pl.BlockSpec((_BLK, _H), lambda g: (g, 0)),
    ],
    out_shape=[
        jax.ShapeDtypeStruct((_N, _H), jnp.float32),
        jax.ShapeDtypeStruct((_N, _H), jnp.float32),
    ],
)


def _layer_a_body(a_ref, b_ref, id_ref, w_ref, bias_ref, z_ref, s1_ref, s2_ref):
    inv_in = lax.rsqrt(jnp.maximum(id_ref[...], 1.0))
    agg = (a_ref[...] + b_ref[...]) * inv_in
    z = jnp.dot(agg, w_ref[...], preferred_element_type=jnp.float32) + bias_ref[...]
    z_ref[...] = z
    s1_ref[...] = jnp.broadcast_to(jnp.sum(z, axis=0, keepdims=True), (8, _H))
    s2_ref[...] = jnp.broadcast_to(jnp.sum(z * z, axis=0, keepdims=True), (8, _H))


_layer_a_call = pl.pallas_call(
    _layer_a_body,
    grid=(_G,),
    in_specs=[
        pl.BlockSpec((_BLK, _H), lambda g: (g, 0)),
        pl.BlockSpec((_BLK, _H), lambda g: (g, 0)),
        pl.BlockSpec((_BLK, 1), lambda g: (g, 0)),
        pl.BlockSpec((_H, _H), lambda g: (0, 0)),
        pl.BlockSpec((1, _H), lambda g: (0, 0)),
    ],
    out_specs=[
        pl.BlockSpec((_BLK, _H), lambda g: (g, 0)),
        pl.BlockSpec((8, _H), lambda g: (g, 0)),
        pl.BlockSpec((8, _H), lambda g: (g, 0)),
    ],
    out_shape=[
        jax.ShapeDtypeStruct((_N, _H), jnp.float32),
        jax.ShapeDtypeStruct((_G * 8, _H), jnp.float32),
        jax.ShapeDtypeStruct((_G * 8, _H), jnp.float32),
    ],
)


def _layer_b_body(z_ref, hin_ref, s1_ref, s2_ref, g_ref, be_ref, od_ref, h_ref, hs_ref):
    mu = jnp.sum(s1_ref[...], axis=0, keepdims=True) * (1.0 / (_N * 8))
    ex2 = jnp.sum(s2_ref[...], axis=0, keepdims=True) * (1.0 / (_N * 8))
    var = ex2 - mu * mu
    zn = (z_ref[...] - mu) * lax.rsqrt(var + 1e-5) * g_ref[...] + be_ref[...]
    hnew = hin_ref[...] + jnp.maximum(zn, 0.0)
    h_ref[...] = hnew
    hs_ref[...] = hnew * lax.rsqrt(jnp.maximum(od_ref[...], 1.0))


_layer_b_call = pl.pallas_call(
    _layer_b_body,
    grid=(_G,),
    in_specs=[
        pl.BlockSpec((_BLK, _H), lambda g: (g, 0)),
        pl.BlockSpec((_BLK, _H), lambda g: (g, 0)),
        pl.BlockSpec((_G * 8, _H), lambda g: (0, 0)),
        pl.BlockSpec((_G * 8, _H), lambda g: (0, 0)),
        pl.BlockSpec((1, _H), lambda g: (0, 0)),
        pl.BlockSpec((1, _H), lambda g: (0, 0)),
        pl.BlockSpec((_BLK, 1), lambda g: (g, 0)),
    ],
    out_specs=[
        pl.BlockSpec((_BLK, _H), lambda g: (g, 0)),
        pl.BlockSpec((_BLK, _H), lambda g: (g, 0)),
    ],
    out_shape=[
        jax.ShapeDtypeStruct((_N, _H), jnp.float32),
        jax.ShapeDtypeStruct((_N, _H), jnp.float32),
    ],
)


def _readout_body(h3_ref, w0_ref, b0_ref, w1_ref, b1_ref, w2_ref, b2_ref, out_ref):
    hg = jnp.mean(h3_ref[...], axis=1)
    x = jnp.maximum(
        jnp.dot(hg, w0_ref[...], preferred_element_type=jnp.float32) + b0_ref[...], 0.0
    )
    x = jnp.maximum(
        jnp.dot(x, w1_ref[...], preferred_element_type=jnp.float32) + b1_ref[...], 0.0
    )
    out_ref[...] = (
        jnp.dot(x, w2_ref[...], preferred_element_type=jnp.float32) + b2_ref[...]
    )


_readout_call = pl.pallas_call(
    _readout_body,
    out_shape=jax.ShapeDtypeStruct((_B, 2), jnp.float32),
)


def kernel(h, e, batch_llms, edge_index, W_h, b_h, pe_emb, W_pe, b_pe, Wl, bl,
           gamma, beta, W1, b1, Wg, bg, Wr0, br0, Wr1, br1, Wr2, br2):
    ei = edge_index.astype(jnp.int32)
    dst_src = jnp.stack([ei[1], ei[0]]).reshape(_NC, _NS, _NCHD, _KD)
    deg = _deg_call(dst_src)
    in_deg = deg[:_N].reshape(_N, 1)
    out_deg = deg[_NPAD:_NPAD + _N].reshape(_N, 1)

    pe = _pe_call(pe_emb, W_pe, b_pe.reshape(1, _H))
    pef = jnp.broadcast_to(pe[None], (_B, _NN, _H)).reshape(_N, _H)
    hcur, hs = _prep_call(h, pef, W_h, b_h.reshape(1, _H), out_deg)

    src = ei[0].reshape(_NW, _NCH, _K)
    dst = ei[1].reshape(_NW, _NCH, _K)
    zeros_pad = jnp.zeros((_NPAD, _H), jnp.float32)
    for l in range(_L):
        agg = _spmm_call(hs, src, dst, zeros_pad)
        z, s1, s2 = _layer_a_call(
            agg[:_N], agg[_NPAD:_NPAD + _N], in_deg, Wl[l], bl[l].reshape(1, _H)
        )
        hcur, hs = _layer_b_call(
            z, hcur, s1, s2,
            gamma[l].reshape(1, _H), beta[l].reshape(1, _H), out_deg,
        )

    return _readout_call(
        hcur.reshape(_B, _NN, _H),
        Wr0, br0.reshape(1, -1),
        Wr1, br1.reshape(1, -1),
        Wr2, br2.reshape(1, -1),
    )


# speedup vs baseline: 11.2084x; 1.2459x over previous
import functools

import jax
import jax.numpy as jnp
from jax import lax
from jax.experimental import pallas as pl
from jax.experimental.pallas import tpu as pltpu
from jax.experimental.pallas import tpu_sc as plsc

_N = 10000
_E = 320000
_H = 128
_B = 100
_NN = 100
_L = 4
_NC = 2
_NS = 16
_NW = _NC * _NS
_EPT = _E // _NS
_EPW = _E // _NW
_K = 100
_NCH = _EPW // _K
_RPT = _N // _NS
_NPAD = 10240
_HR = _NPAD // 16
_BLK = 2000
_G = _N // _BLK

_mesh = plsc.VectorSubcoreMesh(core_axis_name="c", subcore_axis_name="s")


_KD = 80
_NCHD = _EPT // _KD
_NPT = _NPAD // _NS


@functools.partial(
    pl.kernel,
    out_type=jax.ShapeDtypeStruct((_NC * _NPAD,), jnp.float32),
    mesh=_mesh,
    scratch_types=[
        pltpu.VMEM((_NCHD, _KD), jnp.int32),
        pltpu.VMEM((_KD,), jnp.float32),
        pltpu.VMEM((_NPT,), jnp.float32),
        pltpu.VMEM_SHARED((_NPAD,), jnp.float32),
    ],
)
def _deg_call(dst_src_hbm, deg_hbm, idx_v, ones_v, zb_v, acc_sh):
    c = lax.axis_index("c")
    s = lax.axis_index("s")
    zeros16 = jnp.zeros((16,), jnp.float32)
    ones16 = jnp.ones((16,), jnp.float32)

    def zb_body(i, _):
        zb_v[pl.ds(i * 16, 16)] = zeros16
        return 0

    lax.fori_loop(0, _NPT // 16, zb_body, 0)

    def ob_body(i, _):
        ones_v[pl.ds(i * 16, 16)] = ones16
        return 0

    lax.fori_loop(0, _KD // 16, ob_body, 0)

    pltpu.sync_copy(zb_v, acc_sh.at[pl.ds(s * _NPT, _NPT)])
    pltpu.sync_copy(dst_src_hbm.at[c, s], idx_v)
    plsc.subcore_barrier()

    def body(j, _):
        pltpu.sync_copy(ones_v, acc_sh.at[idx_v.at[j]], add=True)
        return 0

    lax.fori_loop(0, _NCHD, body, 0)
    plsc.subcore_barrier()
    base = pl.multiple_of(c * _NPAD + s * _NPT, 128)
    pltpu.sync_copy(
        acc_sh.at[pl.ds(s * _NPT, _NPT)],
        deg_hbm.at[pl.ds(base, _NPT)],
    )


_NBUF = 2
_RQ = _N // _NS - 1


@functools.partial(
    pl.kernel,
    out_type=jax.ShapeDtypeStruct((_NC, _N, _H), jnp.float32),
    mesh=_mesh,
    scratch_types=[
        pltpu.VMEM((_NCH, _K), jnp.int32),
        pltpu.VMEM((_NBUF, 1, _K), jnp.int32),
        pltpu.VMEM((_NBUF * _K, _H), jnp.float32),
        pltpu.SemaphoreType.DMA,
        pltpu.SemaphoreType.DMA,
        pltpu.SemaphoreType.DMA,
        pltpu.VMEM_SHARED((_N, _H), jnp.float32),
    ],
)
def _spmm_call(hs_hbm, src_hbm, dst_hbm, zero_hbm, out_hbm,
               dstv, sbuf, rows_v, gs0, gs1, ss, agg_sh):
    c = lax.axis_index("c")
    s = lax.axis_index("s")
    wid = c * _NS + s
    gsems = (gs0, gs1)
    pltpu.sync_copy(zero_hbm.at[pl.ds(s * _RQ, _RQ)], agg_sh.at[pl.ds(s * _RQ, _RQ)])

    @pl.when(s == 0)
    def _():
        pltpu.sync_copy(
            zero_hbm.at[pl.ds(_RQ * _NS, _N - _RQ * _NS)],
            agg_sh.at[pl.ds(_RQ * _NS, _N - _RQ * _NS)],
        )

    pltpu.sync_copy(dst_hbm.at[wid], dstv)
    for b in range(_NBUF):
        pltpu.sync_copy(src_hbm.at[wid, b], sbuf.at[b])
    plsc.subcore_barrier()

    def body(i, _):
        j = i * _NBUF
        g = [
            pltpu.async_copy(
                hs_hbm.at[sbuf.at[b, 0]], rows_v.at[pl.ds(b * _K, _K)], gsems[b]
            )
            for b in range(_NBUF)
        ]
        sc = []
        for b in range(_NBUF):
            g[b].wait()
            sc.append(
                pltpu.async_copy(
                    rows_v.at[pl.ds(b * _K, _K)],
                    agg_sh.at[dstv.at[j + b]],
                    ss,
                    add=True,
                )
            )

        @pl.when(j + _NBUF < _NCH)
        def _():
            for b in range(_NBUF):
                pltpu.sync_copy(src_hbm.at[wid, j + _NBUF + b], sbuf.at[b])

        for b in range(_NBUF):
            sc[b].wait()
        return 0

    lax.fori_loop(0, _NCH // _NBUF, body, 0)
    plsc.subcore_barrier()
    pltpu.sync_copy(agg_sh.at[pl.ds(s * _RQ, _RQ)], out_hbm.at[c, pl.ds(s * _RQ, _RQ)])

    @pl.when(s == 0)
    def _():
        pltpu.sync_copy(
            agg_sh.at[pl.ds(_RQ * _NS, _N - _RQ * _NS)],
            out_hbm.at[c, pl.ds(_RQ * _NS, _N - _RQ * _NS)],
        )


def _pe_body(pe_emb_ref, wpe_ref, bpe_ref, out_ref):
    out_ref[...] = (
        jnp.dot(pe_emb_ref[...], wpe_ref[...], preferred_element_type=jnp.float32)
        + bpe_ref[...]
    )


_pe_call = pl.pallas_call(
    _pe_body,
    out_shape=jax.ShapeDtypeStruct((_NN, _H), jnp.float32),
)


def _prep_body(h_ref, pef_ref, wh_ref, bh_ref, od_ref, h0_ref, hs_ref):
    h0 = (
        jnp.dot(h_ref[...], wh_ref[...], preferred_element_type=jnp.float32)
        + bh_ref[...]
        + pef_ref[...]
    )
    inv_out = lax.rsqrt(jnp.maximum(od_ref[...], 1.0))
    h0_ref[...] = h0
    hs_ref[...] = h0 * inv_out


_prep_call = pl.pallas_call(
    _prep_body,
    grid=(_G,),
    in_specs=[
        pl.BlockSpec((_BLK, _H), lambda g: (g, 0)),
        pl.BlockSpec((_BLK, _H), lambda g: (g, 0)),
        pl.BlockSpec((_H, _H), lambda g: (0, 0)),
        pl.BlockSpec((1, _H), lambda g: (0, 0)),
        pl.BlockSpec((_BLK, 1), lambda g: (g, 0)),
    ],
    out_specs=[
        pl.BlockSpec((_BLK, _H), lambda g: (g, 0)),
        pl.BlockSpec((_BLK, _H), lambda g: (g, 0)),
    ],
    out_shape=[
        jax.ShapeDtypeStruct((_N, _H), jnp.float32),
        jax.ShapeDtypeStruct((_N, _H), jnp.float32),
    ],
)


def _layer_body(a_ref, b_ref, id_ref, w_ref, bias_ref, hin_ref, gm_ref, be_ref,
                od_ref, h_ref, hs_ref, z_scr, s1_scr, s2_scr):
    ph = pl.program_id(0)
    g = pl.program_id(1)

    @pl.when(ph == 0)
    def _():
        inv_in = lax.rsqrt(jnp.maximum(id_ref[...], 1.0))
        agg = (a_ref[...] + b_ref[...]) * inv_in
        z = (
            jnp.dot(agg, w_ref[...], preferred_element_type=jnp.float32)
            + bias_ref[...]
        )
        z_scr[pl.ds(g * _BLK, _BLK), :] = z

        @pl.when(g == 0)
        def _():
            s1_scr[...] = jnp.zeros((1, _H), jnp.float32)
            s2_scr[...] = jnp.zeros((1, _H), jnp.float32)

        s1_scr[...] += jnp.sum(z, axis=0, keepdims=True)
        s2_scr[...] += jnp.sum(z * z, axis=0, keepdims=True)

    @pl.when(ph == 1)
    def _():
        mu = s1_scr[...] * (1.0 / _N)
        var = s2_scr[...] * (1.0 / _N) - mu * mu
        z = z_scr[pl.ds(g * _BLK, _BLK), :]
        zn = (z - mu) * lax.rsqrt(var + 1e-5) * gm_ref[...] + be_ref[...]
        hnew = hin_ref[...] + jnp.maximum(zn, 0.0)
        h_ref[...] = hnew
        hs_ref[...] = hnew * lax.rsqrt(jnp.maximum(od_ref[...], 1.0))


_layer_call = pl.pallas_call(
    _layer_body,
    grid=(2, _G),
    in_specs=[
        pl.BlockSpec((_BLK, _H), lambda ph, g: (g * (1 - ph), 0)),
        pl.BlockSpec((_BLK, _H), lambda ph, g: (g * (1 - ph), 0)),
        pl.BlockSpec((_BLK, 1), lambda ph, g: (g * (1 - ph), 0)),
        pl.BlockSpec((_H, _H), lambda ph, g: (0, 0)),
        pl.BlockSpec((1, _H), lambda ph, g: (0, 0)),
        pl.BlockSpec((_BLK, _H), lambda ph, g: (g * ph, 0)),
        pl.BlockSpec((1, _H), lambda ph, g: (0, 0)),
        pl.BlockSpec((1, _H), lambda ph, g: (0, 0)),
        pl.BlockSpec((_BLK, 1), lambda ph, g: (g * ph, 0)),
    ],
    out_specs=[
        pl.BlockSpec((_BLK, _H), lambda ph, g: (g * ph, 0)),
        pl.BlockSpec((_BLK, _H), lambda ph, g: (g * ph, 0)),
    ],
    out_shape=[
        jax.ShapeDtypeStruct((_N, _H), jnp.float32),
        jax.ShapeDtypeStruct((_N, _H), jnp.float32),
    ],
    scratch_shapes=[
        pltpu.VMEM((_N, _H), jnp.float32),
        pltpu.VMEM((1, _H), jnp.float32),
        pltpu.VMEM((1, _H), jnp.float32),
    ],
)


def _readout_body(h3_ref, w0_ref, b0_ref, w1_ref, b1_ref, w2_ref, b2_ref, out_ref):
    hg = jnp.mean(h3_ref[...], axis=1)
    x = jnp.maximum(
        jnp.dot(hg, w0_ref[...], preferred_element_type=jnp.float32) + b0_ref[...], 0.0
    )
    x = jnp.maximum(
        jnp.dot(x, w1_ref[...], preferred_element_type=jnp.float32) + b1_ref[...], 0.0
    )
    out_ref[...] = (
        jnp.dot(x, w2_ref[...], preferred_element_type=jnp.float32) + b2_ref[...]
    )


_readout_call = pl.pallas_call(
    _readout_body,
    out_shape=jax.ShapeDtypeStruct((_B, 2), jnp.float32),
)


def kernel(h, e, batch_llms, edge_index, W_h, b_h, pe_emb, W_pe, b_pe, Wl, bl,
           gamma, beta, W1, b1, Wg, bg, Wr0, br0, Wr1, br1, Wr2, br2):
    ei = edge_index.astype(jnp.int32)
    dst_src = jnp.stack([ei[1], ei[0]]).reshape(_NC, _NS, _NCHD, _KD)
    deg = _deg_call(dst_src)
    in_deg = deg[:_N].reshape(_N, 1)
    out_deg = deg[_NPAD:_NPAD + _N].reshape(_N, 1)

    pe = _pe_call(pe_emb, W_pe, b_pe.reshape(1, _H))
    pef = jnp.broadcast_to(pe[None], (_B, _NN, _H)).reshape(_N, _H)
    hcur, hs = _prep_call(h, pef, W_h, b_h.reshape(1, _H), out_deg)

    src = ei[0].reshape(_NW, _NCH, 1, _K)
    dst = ei[1].reshape(_NW, _NCH, _K)
    zeros_n = jnp.zeros((_N, _H), jnp.float32)
    for l in range(_L):
        agg = _spmm_call(hs, src, dst, zeros_n)
        hcur, hs = _layer_call(
            agg[0], agg[1], in_deg, Wl[l], bl[l].reshape(1, _H),
            hcur, gamma[l].reshape(1, _H), beta[l].reshape(1, _H), out_deg,
        )

    return _readout_call(
        hcur.reshape(_B, _NN, _H),
        Wr0, br0.reshape(1, -1),
        Wr1, br1.reshape(1, -1),
        Wr2, br2.reshape(1, -1),
    )
